# Initial kernel scaffold; baseline (speedup 1.0000x reference)
import jax
import jax.numpy as jnp
from jax.experimental import pallas as pl


def kernel(loc, edges, W1i, b1i, W1p, b1p, D1, W2i, b2i, W2p, b2p, D2, Wo, bo):
    raise NotImplementedError("write your pallas kernel here")


# speedup vs baseline: 56.5427x; 56.5427x over previous
import functools

import jax
import jax.numpy as jnp
from jax import lax
from jax.experimental import pallas as pl
from jax.experimental.pallas import tpu as pltpu
from jax.experimental.pallas import tpu_sc as plsc

EPS = 1e-6
CH = 128
NSUB = 16
NCORE = 2


def _mesh():
    return plsc.VectorSubcoreMesh(core_axis_name="c", subcore_axis_name="s")


def _chunked_rows(n, sid, fn_full, fn_tail):
    nfull = n // 128
    tail = n - nfull * 128
    nj = (nfull + NSUB) // NSUB

    def body(j, _):
        k = sid + j * NSUB

        @pl.when(k < nfull)
        def _():
            fn_full(k * 128)

        if tail:
            @pl.when(k == nfull)
            def _():
                fn_tail(nfull * 128)
        return 0

    lax.fori_loop(0, nj, body, 0)


def _zero_acc(acc, zbuf, sid, n):
    _chunked_rows(
        n, sid,
        lambda off: pltpu.sync_copy(zbuf, acc.at[pl.ds(off, 128)]),
        lambda off: pltpu.sync_copy(zbuf.at[pl.ds(0, n - (n // 128) * 128)],
                                    acc.at[pl.ds(off, n - (n // 128) * 128)]))


def _edge_pass(tab, ep, acc, ebuf, rows, gsem, base, nch):
    def fire(k, slot):
        pltpu.sync_copy(ep.at[k], ebuf.at[slot])
        pltpu.async_copy(tab.at[ebuf.at[slot, 0]], rows.at[slot],
                         gsem.at[slot])

    fire(base, 0)

    def eb(i, _):
        slot = lax.rem(i, 2)
        nslot = 1 - slot

        @pl.when(i + 1 < nch)
        def _():
            fire(base + i + 1, nslot)

        pltpu.make_async_copy(tab.at[ebuf.at[slot, 0]], rows.at[slot],
                              gsem.at[slot]).wait()
        pltpu.sync_copy(rows.at[slot], acc.at[ebuf.at[slot, 1]], add=True)
        return 0

    lax.fori_loop(0, nch, eb, 0)


def _writeback(acc, out, bb, sid, n):
    tail = n - (n // 128) * 128

    def full(off):
        pltpu.sync_copy(acc.at[pl.ds(off, 128)], bb)
        pltpu.sync_copy(bb, out.at[pl.ds(off, 128)])

    def part(off):
        pltpu.sync_copy(acc.at[pl.ds(off, tail)], bb.at[pl.ds(0, tail)])
        pltpu.sync_copy(bb.at[pl.ds(0, tail)], out.at[pl.ds(off, tail)])

    _chunked_rows(n, sid, full, part)


def _make_pool1(n, nch1):
    @functools.partial(
        pl.kernel,
        out_type=jax.ShapeDtypeStruct((NCORE, n, 16), jnp.float32),
        mesh=_mesh(),
        compiler_params=pltpu.CompilerParams(use_tc_tiling_on_sc=False),
        scratch_types=[
            pltpu.VMEM((2, 2, CH), jnp.int32),
            pltpu.VMEM((2, CH, 16), jnp.float32),
            pltpu.VMEM((128, 16), jnp.float32),
            pltpu.VMEM((128, 16), jnp.float32),
            pltpu.VMEM_SHARED((n + 8, 16), jnp.float32),
            pltpu.SemaphoreType.DMA((2,)),
        ],
    )
    def pool1(u16, ep, zhbm, out, ebuf, rows, bb, zbuf, acc, gsem):
        cid = lax.axis_index("c")
        sid = lax.axis_index("s")
        pltpu.sync_copy(zhbm, zbuf)
        _zero_acc(acc, zbuf, sid, n)
        plsc.subcore_barrier()
        base = (cid * NSUB + sid) * nch1
        _edge_pass(u16, ep, acc, ebuf, rows, gsem, base, nch1)
        plsc.subcore_barrier()
        for c in range(NCORE):
            @pl.when(cid == c)
            def _(c=c):
                _writeback(acc, out.at[c], bb, sid, n)
        plsc.subcore_barrier()

    return pool1


def _make_pool2(n, nch2):
    sds = jax.ShapeDtypeStruct((n, 32), jnp.float32)

    @functools.partial(
        pl.kernel,
        out_type=(sds,) * 6,
        mesh=_mesh(),
        compiler_params=pltpu.CompilerParams(use_tc_tiling_on_sc=False),
        scratch_types=[
            pltpu.VMEM((2, 2, CH), jnp.int32),
            pltpu.VMEM((2, CH, 32), jnp.float32),
            pltpu.VMEM((128, 32), jnp.float32),
            pltpu.VMEM((128, 32), jnp.float32),
            pltpu.VMEM_SHARED((n + 8, 32), jnp.float32),
            pltpu.SemaphoreType.DMA((2,)),
        ],
    )
    def pool2(x0, x1, x2, x3, x4, x5, ep, zhbm,
              o0, o1, o2, o3, o4, o5, ebuf, rows, bb, zbuf, acc, gsem):
        cid = lax.axis_index("c")
        sid = lax.axis_index("s")
        pltpu.sync_copy(zhbm, zbuf)
        tabs = ((x0, o0), (x1, o1), (x2, o2), (x3, o3), (x4, o4), (x5, o5))
        for c in range(NCORE):
            @pl.when(cid == c)
            def _(c=c):
                for t in range(3):
                    tab, out = tabs[3 * c + t]
                    _zero_acc(acc, zbuf, sid, n)
                    plsc.subcore_barrier()
                    _edge_pass(tab, ep, acc, ebuf, rows, gsem,
                               sid * nch2, nch2)
                    plsc.subcore_barrier()
                    _writeback(acc, out, bb, sid, n)
                    plsc.subcore_barrier()

    return pool2


def _l1_body(u_ref, p_ref, coef_ref, o0, o1, o2, o3, o4, o5):
    u = u_ref[:, 0:3]
    v = p_ref[0, :, 0:3] + p_ref[1, :, 0:3]
    wi = coef_ref[0:1, :]
    wp = coef_ref[1:2, :]
    bs = coef_ref[2:3, :]
    qi = coef_ref[3:4, :]
    qp = coef_ref[4:5, :]
    qb = coef_ref[5:6, :]
    uu = jnp.sum(u * u, 1, keepdims=True)
    uv = jnp.sum(u * v, 1, keepdims=True)
    vv = jnp.sum(v * v, 1, keepdims=True)
    us = jnp.sum(u, 1, keepdims=True)
    vs = jnp.sum(v, 1, keepdims=True)
    dot = (uu * (wi * qi) + uv * (wi * qp + wp * qi) + vv * (wp * qp)
           + us * (wi * qb + bs * qi) + vs * (wp * qb + bs * qp)
           + 3.0 * bs * qb)
    dns = (uu * qi * qi + 2.0 * uv * qi * qp + vv * qp * qp
           + 2.0 * us * qi * qb + 2.0 * vs * qp * qb + 3.0 * qb * qb)
    c = 0.8 * jnp.where(dot < 0, dot / (dns + EPS), 0.0)
    a1 = (1.0 + wi) - c * qi
    a2 = wp - c * qp
    a3 = bs - c * qb
    outs = ((o0, o1), (o2, o3), (o4, o5))
    for j in range(3):
        f = a1 * u[:, j:j + 1] + a2 * v[:, j:j + 1] + a3
        outs[j][0][...] = f[:, :32]
        outs[j][1][...] = f[:, 32:]


def _l2_body(s0, s1, s2, s3, s4, s5, q0, q1, q2, q3, q4, q5,
             w2it, w2pt, d2t, b2, out_ref):
    sl = (s0, s1, s2, s3, s4, s5)
    ql = (q0, q1, q2, q3, q4, q5)
    x1 = [jnp.concatenate([sl[2 * j][...], sl[2 * j + 1][...]], axis=1)
          for j in range(3)]
    p = [jnp.concatenate([ql[2 * j][...], ql[2 * j + 1][...]], axis=1)
         for j in range(3)]
    b2row = b2[0:1, :]
    emb = [jnp.dot(x1[j], w2it[...], preferred_element_type=jnp.float32)
           + jnp.dot(p[j], w2pt[...], preferred_element_type=jnp.float32)
           + b2row for j in range(3)]
    d = [jnp.dot(emb[j], d2t[...], preferred_element_type=jnp.float32)
         for j in range(3)]
    dot = emb[0] * d[0] + emb[1] * d[1] + emb[2] * d[2]
    dns = d[0] * d[0] + d[1] * d[1] + d[2] * d[2]
    c = 0.8 * jnp.where(dot < 0, dot / (dns + EPS), 0.0)
    rows = [jnp.sum(emb[j] - c * d[j] + x1[j], axis=0) for j in range(3)]
    res = jnp.stack(rows, axis=0)

    @pl.when(pl.program_id(0) == 0)
    def _():
        out_ref[...] = res

    @pl.when(pl.program_id(0) != 0)
    def _():
        out_ref[...] = out_ref[...] + res


def kernel(loc, edges, W1i, b1i, W1p, b1p, D1, W2i, b2i, W2p, b2p, D2, Wo, bo):
    n, _ = loc.shape
    e = edges.shape[1]

    mean_loc = jnp.mean(loc, axis=0)
    u = loc - mean_loc
    u16 = jnp.pad(u, ((0, 0), (0, 13)))

    nch1 = (((e + 31) // 32 + CH - 1) // CH)
    nch2 = (((e + 15) // 16 + CH - 1) // CH)
    lp = max(32 * nch1, 16 * nch2) * CH
    srcp = jnp.zeros((lp,), jnp.int32).at[:e].set(edges[0])
    dstp = jnp.full((lp,), n, jnp.int32).at[:e].set(edges[1])
    edges_p = jnp.stack([srcp.reshape(-1, CH), dstp.reshape(-1, CH)], axis=1)

    z16 = jnp.zeros((128, 16), jnp.float32)
    z32 = jnp.zeros((128, 32), jnp.float32)

    wi, wp = W1i[:, 0], W1p[:, 0]
    bs = b1i + b1p
    coef = jnp.zeros((8, 64), jnp.float32)
    coef = coef.at[0].set(wi).at[1].set(wp).at[2].set(bs)
    coef = coef.at[3].set(D1 @ wi).at[4].set(D1 @ wp).at[5].set(D1 @ bs)

    partials = _make_pool1(n, nch1)(u16, edges_p, z16)

    bn = 2000
    grid = (n // bn,)
    sds32 = jax.ShapeDtypeStruct((n, 32), jnp.float32)
    slabs = pl.pallas_call(
        _l1_body,
        grid=grid,
        in_specs=[
            pl.BlockSpec((bn, 16), lambda i: (i, 0)),
            pl.BlockSpec((2, bn, 16), lambda i: (0, i, 0)),
            pl.BlockSpec((8, 64), lambda i: (0, 0)),
        ],
        out_specs=[pl.BlockSpec((bn, 32), lambda i: (i, 0))] * 6,
        out_shape=[sds32] * 6,
    )(u16, partials, coef)

    pooled = _make_pool2(n, nch2)(*slabs, edges_p, z32)

    b2 = jnp.zeros((8, 64), jnp.float32).at[0].set(b2i + b2p)
    s = pl.pallas_call(
        _l2_body,
        grid=grid,
        in_specs=(
            [pl.BlockSpec((bn, 32), lambda i: (i, 0))] * 12
            + [pl.BlockSpec((64, 64), lambda i: (0, 0))] * 3
            + [pl.BlockSpec((8, 64), lambda i: (0, 0))]
        ),
        out_specs=pl.BlockSpec((3, 64), lambda i: (0, 0)),
        out_shape=jax.ShapeDtypeStruct((3, 64), jnp.float32),
    )(*slabs, *pooled, W2i.T, W2p.T, D2.T, b2)

    out = (s @ Wo.T + bo).reshape(1, 3, 4)
    rot = out[:, :, :3]
    trans = out[:, :, 3:] + mean_loc[None, :, None]
    return (rot, jnp.squeeze(trans))


# speedup vs baseline: 78.6846x; 1.3916x over previous
import functools

import jax
import jax.numpy as jnp
from jax import lax
from jax.experimental import pallas as pl
from jax.experimental.pallas import tpu as pltpu
from jax.experimental.pallas import tpu_sc as plsc

EPS = 1e-6
CH = 128
NSUB = 16
NCORE = 2


def _mesh():
    return plsc.VectorSubcoreMesh(core_axis_name="c", subcore_axis_name="s")


def _chunked_rows(n, sid, fn_full, fn_tail):
    nfull = n // 128
    tail = n - nfull * 128
    nj = (nfull + NSUB) // NSUB

    def body(j, _):
        k = sid + j * NSUB

        @pl.when(k < nfull)
        def _():
            fn_full(k * 128)

        if tail:
            @pl.when(k == nfull)
            def _():
                fn_tail(nfull * 128)
        return 0

    lax.fori_loop(0, nj, body, 0)


def _zero_acc(acc, zbuf, sid, n):
    _chunked_rows(
        n, sid,
        lambda off: pltpu.sync_copy(zbuf, acc.at[pl.ds(off, 128)]),
        lambda off: pltpu.sync_copy(zbuf.at[pl.ds(0, n - (n // 128) * 128)],
                                    acc.at[pl.ds(off, n - (n // 128) * 128)]))


NBUF = 4


def _edge_pass(tab, ep, acc, ebuf, rows, isem, gsem, ssem, base, nch):
    def fire_idx(j):
        pltpu.async_copy(ep.at[base + j], ebuf.at[lax.rem(j, NBUF)],
                         isem.at[lax.rem(j, NBUF)])

    def fire_gather(j):
        s = lax.rem(j, NBUF)
        pltpu.make_async_copy(ep.at[base + j], ebuf.at[s], isem.at[s]).wait()
        pltpu.async_copy(tab.at[ebuf.at[s, 0]], rows.at[s], gsem.at[s])

    def fire_scatter(j):
        s = lax.rem(j, NBUF)
        pltpu.make_async_copy(tab.at[ebuf.at[s, 0]], rows.at[s],
                              gsem.at[s]).wait()
        pltpu.async_copy(rows.at[s], acc.at[ebuf.at[s, 1]], ssem.at[s],
                         add=True)

    def wait_scatter(j):
        s = lax.rem(j, NBUF)
        pltpu.make_async_copy(rows.at[s], acc.at[ebuf.at[s, 1]],
                              ssem.at[s]).wait()

    fire_idx(0)
    fire_idx(1)
    fire_idx(2)
    fire_gather(0)
    fire_gather(1)

    def eb(i, _):
        @pl.when(i >= 1)
        def _():
            wait_scatter(i - 1)

        @pl.when(i + 3 < nch)
        def _():
            fire_idx(i + 3)

        @pl.when(i + 2 < nch)
        def _():
            fire_gather(i + 2)

        fire_scatter(i)
        return 0

    lax.fori_loop(0, nch, eb, 0)
    wait_scatter(nch - 1)


def _writeback(acc, out, bb, sid, n):
    tail = n - (n // 128) * 128

    def full(off):
        pltpu.sync_copy(acc.at[pl.ds(off, 128)], bb)
        pltpu.sync_copy(bb, out.at[pl.ds(off, 128)])

    def part(off):
        pltpu.sync_copy(acc.at[pl.ds(off, tail)], bb.at[pl.ds(0, tail)])
        pltpu.sync_copy(bb.at[pl.ds(0, tail)], out.at[pl.ds(off, tail)])

    _chunked_rows(n, sid, full, part)


def _make_pool1(n, nch1):
    @functools.partial(
        pl.kernel,
        out_type=jax.ShapeDtypeStruct((NCORE, n, 16), jnp.float32),
        mesh=_mesh(),
        compiler_params=pltpu.CompilerParams(use_tc_tiling_on_sc=False),
        scratch_types=[
            pltpu.VMEM((NBUF, 2, CH), jnp.int32),
            pltpu.VMEM((NBUF, CH, 16), jnp.float32),
            pltpu.VMEM((128, 16), jnp.float32),
            pltpu.VMEM((128, 16), jnp.float32),
            pltpu.VMEM_SHARED((n + 8, 16), jnp.float32),
            pltpu.SemaphoreType.DMA((NBUF,)),
            pltpu.SemaphoreType.DMA((NBUF,)),
            pltpu.SemaphoreType.DMA((NBUF,)),
        ],
    )
    def pool1(u16, ep, zhbm, out, ebuf, rows, bb, zbuf, acc, isem, gsem, ssem):
        cid = lax.axis_index("c")
        sid = lax.axis_index("s")
        pltpu.sync_copy(zhbm, zbuf)
        _zero_acc(acc, zbuf, sid, n)
        plsc.subcore_barrier()
        base = (cid * NSUB + sid) * nch1
        _edge_pass(u16, ep, acc, ebuf, rows, isem, gsem, ssem, base, nch1)
        plsc.subcore_barrier()
        for c in range(NCORE):
            @pl.when(cid == c)
            def _(c=c):
                _writeback(acc, out.at[c], bb, sid, n)
        plsc.subcore_barrier()

    return pool1


def _make_pool2(n, nch2):
    sds = jax.ShapeDtypeStruct((n, 32), jnp.float32)

    @functools.partial(
        pl.kernel,
        out_type=(sds,) * 6,
        mesh=_mesh(),
        compiler_params=pltpu.CompilerParams(use_tc_tiling_on_sc=False),
        scratch_types=[
            pltpu.VMEM((NBUF, 2, CH), jnp.int32),
            pltpu.VMEM((NBUF, CH, 32), jnp.float32),
            pltpu.VMEM((128, 32), jnp.float32),
            pltpu.VMEM((128, 32), jnp.float32),
            pltpu.VMEM_SHARED((n + 8, 32), jnp.float32),
            pltpu.SemaphoreType.DMA((NBUF,)),
            pltpu.SemaphoreType.DMA((NBUF,)),
            pltpu.SemaphoreType.DMA((NBUF,)),
        ],
    )
    def pool2(x0, x1, x2, x3, x4, x5, ep, zhbm,
              o0, o1, o2, o3, o4, o5, ebuf, rows, bb, zbuf, acc,
              isem, gsem, ssem):
        cid = lax.axis_index("c")
        sid = lax.axis_index("s")
        pltpu.sync_copy(zhbm, zbuf)
        tabs = ((x0, o0), (x1, o1), (x2, o2), (x3, o3), (x4, o4), (x5, o5))
        for c in range(NCORE):
            @pl.when(cid == c)
            def _(c=c):
                for t in range(3):
                    tab, out = tabs[3 * c + t]
                    _zero_acc(acc, zbuf, sid, n)
                    plsc.subcore_barrier()
                    _edge_pass(tab, ep, acc, ebuf, rows, isem, gsem, ssem,
                               sid * nch2, nch2)
                    plsc.subcore_barrier()
                    _writeback(acc, out, bb, sid, n)
                    plsc.subcore_barrier()

    return pool2


def _l1_body(u_ref, p_ref, coef_ref, o0, o1, o2, o3, o4, o5):
    u = u_ref[:, 0:3]
    v = p_ref[0, :, 0:3] + p_ref[1, :, 0:3]
    wi = coef_ref[0:1, :]
    wp = coef_ref[1:2, :]
    bs = coef_ref[2:3, :]
    qi = coef_ref[3:4, :]
    qp = coef_ref[4:5, :]
    qb = coef_ref[5:6, :]
    uu = jnp.sum(u * u, 1, keepdims=True)
    uv = jnp.sum(u * v, 1, keepdims=True)
    vv = jnp.sum(v * v, 1, keepdims=True)
    us = jnp.sum(u, 1, keepdims=True)
    vs = jnp.sum(v, 1, keepdims=True)
    dot = (uu * (wi * qi) + uv * (wi * qp + wp * qi) + vv * (wp * qp)
           + us * (wi * qb + bs * qi) + vs * (wp * qb + bs * qp)
           + 3.0 * bs * qb)
    dns = (uu * qi * qi + 2.0 * uv * qi * qp + vv * qp * qp
           + 2.0 * us * qi * qb + 2.0 * vs * qp * qb + 3.0 * qb * qb)
    c = 0.8 * jnp.where(dot < 0, dot / (dns + EPS), 0.0)
    a1 = (1.0 + wi) - c * qi
    a2 = wp - c * qp
    a3 = bs - c * qb
    outs = ((o0, o1), (o2, o3), (o4, o5))
    for j in range(3):
        f = a1 * u[:, j:j + 1] + a2 * v[:, j:j + 1] + a3
        outs[j][0][...] = f[:, :32]
        outs[j][1][...] = f[:, 32:]


def _l2_body(s0, s1, s2, s3, s4, s5, q0, q1, q2, q3, q4, q5,
             w2it, w2pt, d2t, b2, out_ref):
    sl = (s0, s1, s2, s3, s4, s5)
    ql = (q0, q1, q2, q3, q4, q5)
    x1 = [jnp.concatenate([sl[2 * j][...], sl[2 * j + 1][...]], axis=1)
          for j in range(3)]
    p = [jnp.concatenate([ql[2 * j][...], ql[2 * j + 1][...]], axis=1)
         for j in range(3)]
    b2row = b2[0:1, :]
    emb = [jnp.dot(x1[j], w2it[...], preferred_element_type=jnp.float32)
           + jnp.dot(p[j], w2pt[...], preferred_element_type=jnp.float32)
           + b2row for j in range(3)]
    d = [jnp.dot(emb[j], d2t[...], preferred_element_type=jnp.float32)
         for j in range(3)]
    dot = emb[0] * d[0] + emb[1] * d[1] + emb[2] * d[2]
    dns = d[0] * d[0] + d[1] * d[1] + d[2] * d[2]
    c = 0.8 * jnp.where(dot < 0, dot / (dns + EPS), 0.0)
    rows = [jnp.sum(emb[j] - c * d[j] + x1[j], axis=0) for j in range(3)]
    res = jnp.stack(rows, axis=0)

    @pl.when(pl.program_id(0) == 0)
    def _():
        out_ref[...] = res

    @pl.when(pl.program_id(0) != 0)
    def _():
        out_ref[...] = out_ref[...] + res


def kernel(loc, edges, W1i, b1i, W1p, b1p, D1, W2i, b2i, W2p, b2p, D2, Wo, bo):
    n, _ = loc.shape
    e = edges.shape[1]

    mean_loc = jnp.mean(loc, axis=0)
    u = loc - mean_loc
    u16 = jnp.pad(u, ((0, 0), (0, 13)))

    nch1 = (((e + 31) // 32 + CH - 1) // CH)
    nch2 = (((e + 15) // 16 + CH - 1) // CH)
    lp = max(32 * nch1, 16 * nch2) * CH
    srcp = jnp.zeros((lp,), jnp.int32).at[:e].set(edges[0])
    dstp = jnp.full((lp,), n, jnp.int32).at[:e].set(edges[1])
    edges_p = jnp.stack([srcp.reshape(-1, CH), dstp.reshape(-1, CH)], axis=1)

    z16 = jnp.zeros((128, 16), jnp.float32)
    z32 = jnp.zeros((128, 32), jnp.float32)

    wi, wp = W1i[:, 0], W1p[:, 0]
    bs = b1i + b1p
    coef = jnp.zeros((8, 64), jnp.float32)
    coef = coef.at[0].set(wi).at[1].set(wp).at[2].set(bs)
    coef = coef.at[3].set(D1 @ wi).at[4].set(D1 @ wp).at[5].set(D1 @ bs)

    partials = _make_pool1(n, nch1)(u16, edges_p, z16)

    bn = 2000
    grid = (n // bn,)
    sds32 = jax.ShapeDtypeStruct((n, 32), jnp.float32)
    slabs = pl.pallas_call(
        _l1_body,
        grid=grid,
        in_specs=[
            pl.BlockSpec((bn, 16), lambda i: (i, 0)),
            pl.BlockSpec((2, bn, 16), lambda i: (0, i, 0)),
            pl.BlockSpec((8, 64), lambda i: (0, 0)),
        ],
        out_specs=[pl.BlockSpec((bn, 32), lambda i: (i, 0))] * 6,
        out_shape=[sds32] * 6,
    )(u16, partials, coef)

    pooled = _make_pool2(n, nch2)(*slabs, edges_p, z32)

    b2 = jnp.zeros((8, 64), jnp.float32).at[0].set(b2i + b2p)
    s = pl.pallas_call(
        _l2_body,
        grid=grid,
        in_specs=(
            [pl.BlockSpec((bn, 32), lambda i: (i, 0))] * 12
            + [pl.BlockSpec((64, 64), lambda i: (0, 0))] * 3
            + [pl.BlockSpec((8, 64), lambda i: (0, 0))]
        ),
        out_specs=pl.BlockSpec((3, 64), lambda i: (0, 0)),
        out_shape=jax.ShapeDtypeStruct((3, 64), jnp.float32),
    )(*slabs, *pooled, W2i.T, W2p.T, D2.T, b2)

    out = (s @ Wo.T + bo).reshape(1, 3, 4)
    rot = out[:, :, :3]
    trans = out[:, :, 3:] + mean_loc[None, :, None]
    return (rot, jnp.squeeze(trans))


# speedup vs baseline: 87.9089x; 1.1172x over previous
import functools

import jax
import jax.numpy as jnp
from jax import lax
from jax.experimental import pallas as pl
from jax.experimental.pallas import tpu as pltpu
from jax.experimental.pallas import tpu_sc as plsc

EPS = 1e-6
CH = 128
NSUB = 16
NCORE = 2


def _mesh():
    return plsc.VectorSubcoreMesh(core_axis_name="c", subcore_axis_name="s")


def _chunked_rows(n, sid, fn_full, fn_tail):
    nfull = n // 128
    tail = n - nfull * 128
    nj = (nfull + NSUB) // NSUB

    def body(j, _):
        k = sid + j * NSUB

        @pl.when(k < nfull)
        def _():
            fn_full(k * 128)

        if tail:
            @pl.when(k == nfull)
            def _():
                fn_tail(nfull * 128)
        return 0

    lax.fori_loop(0, nj, body, 0)


def _zero_acc(acc, zbuf, sid, n):
    _chunked_rows(
        n, sid,
        lambda off: pltpu.sync_copy(zbuf, acc.at[pl.ds(off, 128)]),
        lambda off: pltpu.sync_copy(zbuf.at[pl.ds(0, n - (n // 128) * 128)],
                                    acc.at[pl.ds(off, n - (n // 128) * 128)]))


NBUF = 4


def _edge_pass(tab, ep, acc, ebuf, rows, isem, gsem, ssem, base, nch):
    def fire_idx(j):
        pltpu.async_copy(ep.at[base + j], ebuf.at[lax.rem(j, NBUF)],
                         isem.at[lax.rem(j, NBUF)])

    def fire_gather(j):
        s = lax.rem(j, NBUF)
        pltpu.make_async_copy(ep.at[base + j], ebuf.at[s], isem.at[s]).wait()
        pltpu.async_copy(tab.at[ebuf.at[s, 0]], rows.at[s], gsem.at[s])

    def fire_scatter(j):
        s = lax.rem(j, NBUF)
        pltpu.make_async_copy(tab.at[ebuf.at[s, 0]], rows.at[s],
                              gsem.at[s]).wait()
        pltpu.async_copy(rows.at[s], acc.at[ebuf.at[s, 1]], ssem.at[s],
                         add=True)

    def wait_scatter(j):
        s = lax.rem(j, NBUF)
        pltpu.make_async_copy(rows.at[s], acc.at[ebuf.at[s, 1]],
                              ssem.at[s]).wait()

    fire_idx(0)
    fire_idx(1)
    fire_idx(2)
    fire_gather(0)
    fire_gather(1)

    def eb(i, _):
        @pl.when(i >= 1)
        def _():
            wait_scatter(i - 1)

        @pl.when(i + 3 < nch)
        def _():
            fire_idx(i + 3)

        @pl.when(i + 2 < nch)
        def _():
            fire_gather(i + 2)

        fire_scatter(i)
        return 0

    lax.fori_loop(0, nch, eb, 0)
    wait_scatter(nch - 1)


def _writeback(acc, out, bb, sid, n):
    tail = n - (n // 128) * 128

    def full(off):
        pltpu.sync_copy(acc.at[pl.ds(off, 128)], bb)
        pltpu.sync_copy(bb, out.at[pl.ds(off, 128)])

    def part(off):
        pltpu.sync_copy(acc.at[pl.ds(off, tail)], bb.at[pl.ds(0, tail)])
        pltpu.sync_copy(bb.at[pl.ds(0, tail)], out.at[pl.ds(off, tail)])

    _chunked_rows(n, sid, full, part)


def _make_pool1(n, nch1):
    @functools.partial(
        pl.kernel,
        out_type=jax.ShapeDtypeStruct((NCORE, n, 16), jnp.float32),
        mesh=_mesh(),
        compiler_params=pltpu.CompilerParams(use_tc_tiling_on_sc=False),
        scratch_types=[
            pltpu.VMEM((NBUF, 2, CH), jnp.int32),
            pltpu.VMEM((NBUF, CH, 16), jnp.float32),
            pltpu.VMEM((128, 16), jnp.float32),
            pltpu.VMEM((128, 16), jnp.float32),
            pltpu.VMEM_SHARED((n + 8, 16), jnp.float32),
            pltpu.SemaphoreType.DMA((NBUF,)),
            pltpu.SemaphoreType.DMA((NBUF,)),
            pltpu.SemaphoreType.DMA((NBUF,)),
        ],
    )
    def pool1(u16, ep, zhbm, out, ebuf, rows, bb, zbuf, acc, isem, gsem, ssem):
        cid = lax.axis_index("c")
        sid = lax.axis_index("s")
        pltpu.sync_copy(zhbm, zbuf)
        _zero_acc(acc, zbuf, sid, n)
        plsc.subcore_barrier()
        base = (cid * NSUB + sid) * nch1
        _edge_pass(u16, ep, acc, ebuf, rows, isem, gsem, ssem, base, nch1)
        plsc.subcore_barrier()
        for c in range(NCORE):
            @pl.when(cid == c)
            def _(c=c):
                _writeback(acc, out.at[c], bb, sid, n)
        plsc.subcore_barrier()

    return pool1


def _make_pool2(npad, n, nch2):
    sds = jax.ShapeDtypeStruct((npad, 32), jnp.float32)

    @functools.partial(
        pl.kernel,
        out_type=(sds,) * 6,
        mesh=_mesh(),
        compiler_params=pltpu.CompilerParams(use_tc_tiling_on_sc=False),
        scratch_types=[
            pltpu.VMEM((NBUF, 2, CH), jnp.int32),
            pltpu.VMEM((NBUF, CH, 32), jnp.float32),
            pltpu.VMEM((128, 32), jnp.float32),
            pltpu.VMEM((128, 32), jnp.float32),
            pltpu.VMEM_SHARED((npad, 32), jnp.float32),
            pltpu.SemaphoreType.DMA((NBUF,)),
            pltpu.SemaphoreType.DMA((NBUF,)),
            pltpu.SemaphoreType.DMA((NBUF,)),
        ],
    )
    def pool2(x0, x1, x2, x3, x4, x5, ep, zhbm,
              o0, o1, o2, o3, o4, o5, ebuf, rows, bb, zbuf, acc,
              isem, gsem, ssem):
        cid = lax.axis_index("c")
        sid = lax.axis_index("s")
        pltpu.sync_copy(zhbm, zbuf)
        tabs = ((x0, o0), (x1, o1), (x2, o2), (x3, o3), (x4, o4), (x5, o5))
        for c in range(NCORE):
            @pl.when(cid == c)
            def _(c=c):
                for t in range(3):
                    tab, out = tabs[3 * c + t]
                    _zero_acc(acc, zbuf, sid, npad)
                    plsc.subcore_barrier()
                    _edge_pass(tab, ep, acc, ebuf, rows, isem, gsem, ssem,
                               sid * nch2, nch2)
                    plsc.subcore_barrier()
                    _writeback(acc, out, bb, sid, npad)
                    plsc.subcore_barrier()

    return pool2


def _l1_body(uv_ref, coef_ref, o0, o1, o2, o3, o4, o5):
    wi = coef_ref[0:1, :]
    wp = coef_ref[1:2, :]
    bs = coef_ref[2:3, :]
    qi = coef_ref[3:4, :]
    qp = coef_ref[4:5, :]
    qb = coef_ref[5:6, :]
    f_aj = []
    for a in range(4):
        u = uv_ref[:, 32 * a:32 * a + 3]
        v = uv_ref[:, 32 * a + 16:32 * a + 19]
        uu = jnp.sum(u * u, 1, keepdims=True)
        uv = jnp.sum(u * v, 1, keepdims=True)
        vv = jnp.sum(v * v, 1, keepdims=True)
        us = jnp.sum(u, 1, keepdims=True)
        vs = jnp.sum(v, 1, keepdims=True)
        dot = (uu * (wi * qi) + uv * (wi * qp + wp * qi) + vv * (wp * qp)
               + us * (wi * qb + bs * qi) + vs * (wp * qb + bs * qp)
               + 3.0 * bs * qb)
        dns = (uu * qi * qi + 2.0 * uv * qi * qp + vv * qp * qp
               + 2.0 * us * qi * qb + 2.0 * vs * qp * qb + 3.0 * qb * qb)
        c = 0.8 * jnp.where(dot < 0, dot / (dns + EPS), 0.0)
        a1 = (1.0 + wi) - c * qi
        a2 = wp - c * qp
        a3 = bs - c * qb
        f_aj.append([a1 * u[:, j:j + 1] + a2 * v[:, j:j + 1] + a3
                     for j in range(3)])
    outs = ((o0, o1), (o2, o3), (o4, o5))
    for j in range(3):
        outs[j][0][...] = jnp.concatenate(
            [f_aj[a][j][:, :32] for a in range(4)], axis=1)
        outs[j][1][...] = jnp.concatenate(
            [f_aj[a][j][:, 32:] for a in range(4)], axis=1)


def _interleave_halves(lo, hi):
    parts = []
    for a in range(4):
        parts.append(lo[:, 32 * a:32 * a + 32])
        parts.append(hi[:, 32 * a:32 * a + 32])
    return jnp.concatenate(parts, axis=1)


def _l2_body(nvalid, brows, s0, s1, s2, s3, s4, s5, q0, q1, q2, q3, q4, q5,
             wit_t, wit_b, wpt_t, wpt_b, d2bd, b2t, out_ref):
    sl = (s0, s1, s2, s3, s4, s5)
    ql = (q0, q1, q2, q3, q4, q5)

    def mm(a, b):
        return jnp.dot(a, b, preferred_element_type=jnp.float32)

    gr = (pl.program_id(0) * brows
          + jax.lax.broadcasted_iota(jnp.int32, (brows, 1), 0))
    w = gr < nvalid
    embs, x1p = [], []
    for j in range(3):
        emb = (mm(sl[2 * j][...], wit_t[...]) + mm(sl[2 * j + 1][...], wit_b[...])
               + mm(ql[2 * j][...], wpt_t[...]) + mm(ql[2 * j + 1][...], wpt_b[...])
               + b2t[0:1, :])
        embs.append(emb)
        x1p.append(_interleave_halves(sl[2 * j][...], sl[2 * j + 1][...]))
    d = [mm(embs[j], d2bd[...]) for j in range(3)]
    dot = embs[0] * d[0] + embs[1] * d[1] + embs[2] * d[2]
    dns = d[0] * d[0] + d[1] * d[1] + d[2] * d[2]
    c = 0.8 * jnp.where(dot < 0, dot / (dns + EPS), 0.0)
    rows = [jnp.sum(jnp.where(w, embs[j] - c * d[j] + x1p[j], 0.0), axis=0)
            for j in range(3)]
    res = jnp.stack(rows, axis=0)

    @pl.when(pl.program_id(0) == 0)
    def _():
        out_ref[...] = res

    @pl.when(pl.program_id(0) != 0)
    def _():
        out_ref[...] = out_ref[...] + res


def kernel(loc, edges, W1i, b1i, W1p, b1p, D1, W2i, b2i, W2p, b2p, D2, Wo, bo):
    n, _ = loc.shape
    e = edges.shape[1]

    mean_loc = jnp.mean(loc, axis=0)
    u = loc - mean_loc
    u16 = jnp.pad(u, ((0, 0), (0, 13)))

    nch1 = (((e + 31) // 32 + CH - 1) // CH)
    nch2 = (((e + 15) // 16 + CH - 1) // CH)
    lp = max(32 * nch1, 16 * nch2) * CH
    srcp = jnp.zeros((lp,), jnp.int32).at[:e].set(edges[0])
    dstp = jnp.full((lp,), n, jnp.int32).at[:e].set(edges[1])
    edges_p = jnp.stack([srcp.reshape(-1, CH), dstp.reshape(-1, CH)], axis=1)

    z16 = jnp.zeros((128, 16), jnp.float32)
    z32 = jnp.zeros((128, 32), jnp.float32)

    wi, wp = W1i[:, 0], W1p[:, 0]
    bs = b1i + b1p
    coef = jnp.zeros((8, 64), jnp.float32)
    coef = coef.at[0].set(wi).at[1].set(wp).at[2].set(bs)
    coef = coef.at[3].set(D1 @ wi).at[4].set(D1 @ wp).at[5].set(D1 @ bs)

    partials = _make_pool1(n, nch1)(u16, edges_p, z16)

    npad = ((n + 63) // 64) * 64
    bn = next(npad // g for g in range(16, 64)
              if npad % g == 0 and (npad // g) % 32 == 0)
    grid = (npad // bn,)
    uvcat = jnp.pad(
        jnp.concatenate([u16, partials[0] + partials[1]], axis=1),
        ((0, npad - n), (0, 0))).reshape(npad // 4, 128)
    sds_pk = jax.ShapeDtypeStruct((npad // 4, 128), jnp.float32)
    slabs_pk = pl.pallas_call(
        _l1_body,
        grid=grid,
        in_specs=[
            pl.BlockSpec((bn // 4, 128), lambda i: (i, 0)),
            pl.BlockSpec((8, 64), lambda i: (0, 0)),
        ],
        out_specs=[pl.BlockSpec((bn // 4, 128), lambda i: (i, 0))] * 6,
        out_shape=[sds_pk] * 6,
    )(uvcat, coef)

    slabs = [s.reshape(npad, 32) for s in slabs_pk]
    pooled = _make_pool2(npad, n, nch2)(*slabs, edges_p, z32)
    pooled_pk = [p.reshape(npad // 4, 128) for p in pooled]

    b2t = jnp.tile((b2i + b2p)[None, :], (1, 4))
    eye4 = jnp.eye(4, dtype=jnp.float32)
    w2it, w2pt, d2t = W2i.T, W2p.T, D2.T
    bd = lambda m: jnp.kron(eye4, m)
    s = pl.pallas_call(
        functools.partial(_l2_body, n // 4, bn // 4),
        grid=grid,
        in_specs=(
            [pl.BlockSpec((bn // 4, 128), lambda i: (i, 0))] * 12
            + [pl.BlockSpec((128, 256), lambda i: (0, 0))] * 4
            + [pl.BlockSpec((256, 256), lambda i: (0, 0))]
            + [pl.BlockSpec((1, 256), lambda i: (0, 0))]
        ),
        out_specs=pl.BlockSpec((3, 256), lambda i: (0, 0)),
        out_shape=jax.ShapeDtypeStruct((3, 256), jnp.float32),
    )(*slabs_pk, *pooled_pk,
      bd(w2it[:32, :]), bd(w2it[32:, :]),
      bd(w2pt[:32, :]), bd(w2pt[32:, :]), bd(d2t), b2t)
    s = s.reshape(3, 4, 64).sum(axis=1)

    out = (s @ Wo.T + bo).reshape(1, 3, 4)
    rot = out[:, :, :3]
    trans = out[:, :, 3:] + mean_loc[None, :, None]
    return (rot, jnp.squeeze(trans))
